# baseline (speedup 1.0000x reference)
import functools

import jax
import jax.numpy as jnp
from jax import lax
from jax.experimental import pallas as pl
from jax.experimental.pallas import tpu as pltpu
from jax.experimental.pallas import tpu_sc as plsc

K = 8
L = 16
NUM_CORES = 2
NUM_SUBCORES = 16
NW = NUM_CORES * NUM_SUBCORES


_SORT8 = ((0, 1), (2, 3), (4, 5), (6, 7),
          (0, 2), (1, 3), (4, 6), (5, 7),
          (1, 2), (5, 6),
          (0, 4), (1, 5), (2, 6), (3, 7),
          (2, 4), (3, 5),
          (1, 2), (3, 4), (5, 6))
_BITONIC8 = ((0, 4), (1, 5), (2, 6), (3, 7),
             (0, 2), (1, 3), (4, 6), (5, 7),
             (0, 1), (2, 3), (4, 5), (6, 7))


def _apply_net(net, v):
    v = list(v)
    for a, b in net:
        lo = jnp.minimum(v[a], v[b])
        hi = jnp.maximum(v[a], v[b])
        v[a], v[b] = lo, hi
    return v


def _merge_bottom8(S, C):
    t = [jnp.minimum(S[i], C[K - 1 - i]) for i in range(K)]
    return _apply_net(_BITONIC8, t)


def _bcast_lane(v, j):
    idx = jnp.full((L,), j, dtype=jnp.int32)
    return v.at[idx].get(mode="promise_in_bounds")


def _sc_kernel(num_b, num_i, inp_hbm, w_hbm, out_hbm,
               inp_v, w_v, out_v, pw_v, mw_v, pi_v, mi_v):
    wid = lax.axis_index("s") * NUM_CORES + lax.axis_index("c")
    chunk_w = num_i * L
    chunk_o = num_b * L
    pltpu.sync_copy(inp_hbm, inp_v)
    pltpu.sync_copy(w_hbm.at[pl.ds(wid * chunk_w, chunk_w)], w_v)

    inf = jnp.full((L,), jnp.inf, dtype=jnp.float32)

    def w_body(i, carry):
        w = w_v[pl.ds(i * L, L)]
        pw = jnp.maximum(w, 0.0)
        pw_v[pl.ds(i * L, L)] = pw
        mw_v[pl.ds(i * L, L)] = pw - w
        return carry

    lax.fori_loop(0, num_i, w_body, 0)

    def b_body(b, carry):
        base = b * num_i

        def x_body(k, carry2):
            xv = inp_v[pl.ds(base + k * L, L)]
            pi_v[pl.ds(k * L, L)] = jnp.maximum(xv + 4.0, 0.0)
            mi_v[pl.ds(k * L, L)] = jnp.maximum(4.0 - xv, 0.0)
            return carry2

        lax.fori_loop(0, num_i // L, x_body, 0)

        def blk_body(k, S):
            piv = pi_v[pl.ds(k * L, L)]
            miv = mi_v[pl.ds(k * L, L)]
            Tp, Tm = [], []
            for q in range(L // 4):
                Cp, Cm = [], []
                for j in range(4 * q, 4 * q + 4):
                    pi = _bcast_lane(piv, j)
                    mi = _bcast_lane(miv, j)
                    pw = pw_v[pl.ds((k * L + j) * L, L)]
                    mw = mw_v[pl.ds((k * L + j) * L, L)]
                    Cp += [pi + pw, mi + mw]
                    Cm += [pi + mw, mi + pw]
                Tp.append(_apply_net(_SORT8, Cp))
                Tm.append(_apply_net(_SORT8, Cm))
            Tp1 = _merge_bottom8(Tp[0], Tp[1])
            Tp2 = _merge_bottom8(Tp[2], Tp[3])
            Tm1 = _merge_bottom8(Tm[0], Tm[1])
            Tm2 = _merge_bottom8(Tm[2], Tm[3])
            Sp = _merge_bottom8(list(S[:K]), _merge_bottom8(Tp1, Tp2))
            Sm = _merge_bottom8(list(S[K:]), _merge_bottom8(Tm1, Tm2))
            return tuple(Sp) + tuple(Sm)

        S = lax.fori_loop(0, num_i // L, blk_body, (inf,) * (2 * K))
        SP, SM = S[:K], S[K:]
        resP = SP[0]
        for j in range(1, K):
            resP = resP + SP[j]
        resM = SM[0]
        for j in range(1, K):
            resM = resM + SM[j]
        out_v[pl.ds(b * L, L)] = (resP - resM) * (1.0 / K)
        return carry

    lax.fori_loop(0, num_b, b_body, 0)
    pltpu.sync_copy(out_v, out_hbm.at[pl.ds(wid * chunk_o, chunk_o)])


def _tc_body(x_ref, w_ref, out_ref, pw_s, mw_s, pib_s, mib_s):
    num_i = x_ref.shape[1]
    oj = pl.program_id(1)

    @pl.when(oj == 0)
    def _():
        x = x_ref[...]
        pi_full = jnp.maximum(x + 4.0, 0.0)
        mi_full = jnp.maximum(4.0 - x, 0.0)
        lane = lax.broadcasted_iota(jnp.int32, (8, num_i), 1)

        def build(i, c):
            bp = jnp.max(jnp.where(lane == i, pi_full, -jnp.inf),
                         axis=1, keepdims=True)
            bm = jnp.max(jnp.where(lane == i, mi_full, -jnp.inf),
                         axis=1, keepdims=True)
            pib_s[i] = jnp.broadcast_to(bp, (8, 128))
            mib_s[i] = jnp.broadcast_to(bm, (8, 128))
            return c

        lax.fori_loop(0, num_i, build, 0)

    def wbuild(i, c):
        w = jnp.broadcast_to(w_ref[pl.ds(i, 1), :], (8, 128))
        pw = jnp.maximum(w, 0.0)
        pw_s[i] = pw
        mw_s[i] = pw - w
        return c

    lax.fori_loop(0, num_i, wbuild, 0)

    inf = jnp.full((8, 128), jnp.inf, dtype=jnp.float32)

    def half(k, q0):
        T = []
        for q in (q0, q0 + 1):
            Cp, Cm = [], []
            for j in range(4 * q, 4 * q + 4):
                i = k * L + j
                pi = pib_s[i]
                mi = mib_s[i]
                pw = pw_s[i]
                mw = mw_s[i]
                Cp += [pi + pw, mi + mw]
                Cm += [pi + mw, mi + pw]
            T.append((_apply_net(_SORT8, Cp), _apply_net(_SORT8, Cm)))
        return (_merge_bottom8(T[0][0], T[1][0]),
                _merge_bottom8(T[0][1], T[1][1]))

    def blk_body(k, S):
        Tp1, Tm1 = half(k, 0)
        Tp2, Tm2 = half(k, 2)
        Sp = _merge_bottom8(list(S[:K]), _merge_bottom8(Tp1, Tp2))
        Sm = _merge_bottom8(list(S[K:]), _merge_bottom8(Tm1, Tm2))
        return tuple(Sp) + tuple(Sm)

    S = lax.fori_loop(0, num_i // L, blk_body, (inf,) * (2 * K))
    resP = S[0]
    for j in range(1, K):
        resP = resP + S[j]
    resM = S[K]
    for j in range(1, K):
        resM = resM + S[K + j]
    out_ref[...] = (resP - resM) * (1.0 / K)


def _tc_part(x_tc, weight):
    b_tc, num_i = x_tc.shape
    _, num_o = weight.shape
    return pl.pallas_call(
        _tc_body,
        grid=(b_tc // 8, num_o // 128),
        in_specs=[
            pl.BlockSpec((8, num_i), lambda bi, oj: (bi, 0)),
            pl.BlockSpec((num_i, 128), lambda bi, oj: (0, oj)),
        ],
        out_specs=pl.BlockSpec((8, 128), lambda bi, oj: (bi, oj)),
        out_shape=jax.ShapeDtypeStruct((b_tc, num_o), jnp.float32),
        scratch_shapes=[
            pltpu.VMEM((num_i, 8, 128), jnp.float32),
            pltpu.VMEM((num_i, 8, 128), jnp.float32),
            pltpu.VMEM((num_i, 8, 128), jnp.float32),
            pltpu.VMEM((num_i, 8, 128), jnp.float32),
        ],
    )(x_tc, weight)


B_SC = 72


def kernel(inputp, weight):
    num_b, num_i = inputp.shape
    _, num_o = weight.shape
    b_sc = B_SC if num_b > B_SC else num_b
    if b_sc < num_b:
        out_tc = _tc_part(inputp[b_sc:], weight)
        out_sc = _sc_part(inputp[:b_sc], weight)
        return jnp.concatenate([out_sc, out_tc], axis=0)
    return _sc_part(inputp, weight)


def _sc_part(inputp, weight):
    num_b, num_i = inputp.shape
    _, num_o = weight.shape
    w_chunks = weight.reshape(num_i, NW, L).transpose(1, 0, 2).reshape(-1)
    inp_flat = inputp.reshape(-1)
    mesh = plsc.VectorSubcoreMesh(
        core_axis_name="c", subcore_axis_name="s",
        num_cores=NUM_CORES, num_subcores=NUM_SUBCORES)
    f = pl.kernel(
        functools.partial(_sc_kernel, num_b, num_i),
        out_type=jax.ShapeDtypeStruct((NW * num_b * L,), jnp.float32),
        mesh=mesh,
        scratch_types=[
            pltpu.VMEM((num_b * num_i,), jnp.float32),
            pltpu.VMEM((num_i * L,), jnp.float32),
            pltpu.VMEM((num_b * L,), jnp.float32),
            pltpu.VMEM((num_i * L,), jnp.float32),
            pltpu.VMEM((num_i * L,), jnp.float32),
            pltpu.VMEM((num_i,), jnp.float32),
            pltpu.VMEM((num_i,), jnp.float32),
        ],
    )
    out = f(inp_flat, w_chunks)
    return out.reshape(NW, num_b, L).transpose(1, 0, 2).reshape(num_b, num_o)


# speedup vs baseline: 1.2380x; 1.2380x over previous
import functools

import jax
import jax.numpy as jnp
from jax import lax
from jax.experimental import pallas as pl
from jax.experimental.pallas import tpu as pltpu
from jax.experimental.pallas import tpu_sc as plsc

K = 8
L = 16
NUM_CORES = 2
NUM_SUBCORES = 16
NW = NUM_CORES * NUM_SUBCORES


_SORT8 = ((0, 1), (2, 3), (4, 5), (6, 7),
          (0, 2), (1, 3), (4, 6), (5, 7),
          (1, 2), (5, 6),
          (0, 4), (1, 5), (2, 6), (3, 7),
          (2, 4), (3, 5),
          (1, 2), (3, 4), (5, 6))
_BITONIC8 = ((0, 4), (1, 5), (2, 6), (3, 7),
             (0, 2), (1, 3), (4, 6), (5, 7),
             (0, 1), (2, 3), (4, 5), (6, 7))


def _apply_net(net, v):
    v = list(v)
    for a, b in net:
        lo = jnp.minimum(v[a], v[b])
        hi = jnp.maximum(v[a], v[b])
        v[a], v[b] = lo, hi
    return v


def _merge_bottom8(S, C):
    t = [jnp.minimum(S[i], C[K - 1 - i]) for i in range(K)]
    return _apply_net(_BITONIC8, t)


def _bcast_lane(v, j):
    idx = jnp.full((L,), j, dtype=jnp.int32)
    return v.at[idx].get(mode="promise_in_bounds")


def _sc_kernel(num_b, num_i, inp_hbm, w_hbm, out_hbm,
               inp_v, w_v, out_v, pw_v, mw_v, pi_v, mi_v):
    wid = lax.axis_index("s") * NUM_CORES + lax.axis_index("c")
    chunk_w = num_i * L
    chunk_o = num_b * L
    pltpu.sync_copy(inp_hbm, inp_v)
    pltpu.sync_copy(w_hbm.at[pl.ds(wid * chunk_w, chunk_w)], w_v)

    inf = jnp.full((L,), jnp.inf, dtype=jnp.float32)

    def w_body(i, carry):
        w = w_v[pl.ds(i * L, L)]
        pw = jnp.maximum(w, 0.0)
        pw_v[pl.ds(i * L, L)] = pw
        mw_v[pl.ds(i * L, L)] = pw - w
        return carry

    lax.fori_loop(0, num_i, w_body, 0)

    def b_body(b, carry):
        base = b * num_i

        def x_body(k, carry2):
            xv = inp_v[pl.ds(base + k * L, L)]
            pi_v[pl.ds(k * L, L)] = jnp.maximum(xv + 4.0, 0.0)
            mi_v[pl.ds(k * L, L)] = jnp.maximum(4.0 - xv, 0.0)
            return carry2

        lax.fori_loop(0, num_i // L, x_body, 0)

        def blk_body(k, S):
            piv = pi_v[pl.ds(k * L, L)]
            miv = mi_v[pl.ds(k * L, L)]
            Tp, Tm = [], []
            for q in range(L // 4):
                Cp, Cm = [], []
                for j in range(4 * q, 4 * q + 4):
                    pi = _bcast_lane(piv, j)
                    mi = _bcast_lane(miv, j)
                    pw = pw_v[pl.ds((k * L + j) * L, L)]
                    mw = mw_v[pl.ds((k * L + j) * L, L)]
                    Cp += [pi + pw, mi + mw]
                    Cm += [pi + mw, mi + pw]
                Tp.append(_apply_net(_SORT8, Cp))
                Tm.append(_apply_net(_SORT8, Cm))
            Tp1 = _merge_bottom8(Tp[0], Tp[1])
            Tp2 = _merge_bottom8(Tp[2], Tp[3])
            Tm1 = _merge_bottom8(Tm[0], Tm[1])
            Tm2 = _merge_bottom8(Tm[2], Tm[3])
            Sp = _merge_bottom8(list(S[:K]), _merge_bottom8(Tp1, Tp2))
            Sm = _merge_bottom8(list(S[K:]), _merge_bottom8(Tm1, Tm2))
            return tuple(Sp) + tuple(Sm)

        S = lax.fori_loop(0, num_i // L, blk_body, (inf,) * (2 * K))
        SP, SM = S[:K], S[K:]
        resP = SP[0]
        for j in range(1, K):
            resP = resP + SP[j]
        resM = SM[0]
        for j in range(1, K):
            resM = resM + SM[j]
        out_v[pl.ds(b * L, L)] = (resP - resM) * (1.0 / K)
        return carry

    lax.fori_loop(0, num_b, b_body, 0)
    pltpu.sync_copy(out_v, out_hbm.at[pl.ds(wid * chunk_o, chunk_o)])


def _tc_body(x_ref, w_ref, out_ref, pw_s, mw_s, pib_s, mib_s):
    num_i = x_ref.shape[1]
    oj = pl.program_id(1)

    @pl.when(oj == 0)
    def _():
        x = x_ref[...]
        pi_full = jnp.maximum(x + 4.0, 0.0)
        mi_full = jnp.maximum(4.0 - x, 0.0)
        lane = lax.broadcasted_iota(jnp.int32, (8, num_i), 1)

        def build(i, c):
            bp = jnp.max(jnp.where(lane == i, pi_full, -jnp.inf),
                         axis=1, keepdims=True)
            bm = jnp.max(jnp.where(lane == i, mi_full, -jnp.inf),
                         axis=1, keepdims=True)
            pib_s[i] = jnp.broadcast_to(bp, (8, 128))
            mib_s[i] = jnp.broadcast_to(bm, (8, 128))
            return c

        lax.fori_loop(0, num_i, build, 0)

    w = w_ref[...]
    pw_full = jnp.maximum(w, 0.0)
    pw_s[...] = pw_full
    mw_s[...] = pw_full - w
    inf = jnp.full((8, 128), jnp.inf, dtype=jnp.float32)

    def half(k, q0):
        T = []
        for q in (q0, q0 + 1):
            Cp, Cm = [], []
            for j in range(4 * q, 4 * q + 4):
                i = k * L + j
                pi = pib_s[i]
                mi = mib_s[i]
                pw = jnp.broadcast_to(pw_s[pl.ds(i, 1), :], (8, 128))
                mw = jnp.broadcast_to(mw_s[pl.ds(i, 1), :], (8, 128))
                Cp += [pi + pw, mi + mw]
                Cm += [pi + mw, mi + pw]
            T.append((_apply_net(_SORT8, Cp), _apply_net(_SORT8, Cm)))
        return (_merge_bottom8(T[0][0], T[1][0]),
                _merge_bottom8(T[0][1], T[1][1]))

    def blk_body(k, S):
        Tp1, Tm1 = half(k, 0)
        Tp2, Tm2 = half(k, 2)
        Sp = _merge_bottom8(list(S[:K]), _merge_bottom8(Tp1, Tp2))
        Sm = _merge_bottom8(list(S[K:]), _merge_bottom8(Tm1, Tm2))
        return tuple(Sp) + tuple(Sm)

    S = lax.fori_loop(0, num_i // L, blk_body, (inf,) * (2 * K))
    resP = S[0]
    for j in range(1, K):
        resP = resP + S[j]
    resM = S[K]
    for j in range(1, K):
        resM = resM + S[K + j]
    out_ref[...] = (resP - resM) * (1.0 / K)


def _tc_part(x_tc, weight):
    b_tc, num_i = x_tc.shape
    _, num_o = weight.shape
    return pl.pallas_call(
        _tc_body,
        grid=(b_tc // 8, num_o // 128),
        in_specs=[
            pl.BlockSpec((8, num_i), lambda bi, oj: (bi, 0)),
            pl.BlockSpec((num_i, 128), lambda bi, oj: (0, oj)),
        ],
        out_specs=pl.BlockSpec((8, 128), lambda bi, oj: (bi, oj)),
        out_shape=jax.ShapeDtypeStruct((b_tc, num_o), jnp.float32),
        scratch_shapes=[
            pltpu.VMEM((num_i, 128), jnp.float32),
            pltpu.VMEM((num_i, 128), jnp.float32),
            pltpu.VMEM((num_i, 8, 128), jnp.float32),
            pltpu.VMEM((num_i, 8, 128), jnp.float32),
        ],
    )(x_tc, weight)


B_SC = 72


def kernel(inputp, weight):
    num_b, num_i = inputp.shape
    _, num_o = weight.shape
    b_sc = B_SC if num_b > B_SC else num_b
    if b_sc < num_b:
        out_tc = _tc_part(inputp[b_sc:], weight)
        out_sc = _sc_part(inputp[:b_sc], weight)
        return jnp.concatenate([out_sc, out_tc], axis=0)
    return _sc_part(inputp, weight)


def _sc_part(inputp, weight):
    num_b, num_i = inputp.shape
    _, num_o = weight.shape
    w_chunks = weight.reshape(num_i, NW, L).transpose(1, 0, 2).reshape(-1)
    inp_flat = inputp.reshape(-1)
    mesh = plsc.VectorSubcoreMesh(
        core_axis_name="c", subcore_axis_name="s",
        num_cores=NUM_CORES, num_subcores=NUM_SUBCORES)
    f = pl.kernel(
        functools.partial(_sc_kernel, num_b, num_i),
        out_type=jax.ShapeDtypeStruct((NW * num_b * L,), jnp.float32),
        mesh=mesh,
        scratch_types=[
            pltpu.VMEM((num_b * num_i,), jnp.float32),
            pltpu.VMEM((num_i * L,), jnp.float32),
            pltpu.VMEM((num_b * L,), jnp.float32),
            pltpu.VMEM((num_i * L,), jnp.float32),
            pltpu.VMEM((num_i * L,), jnp.float32),
            pltpu.VMEM((num_i,), jnp.float32),
            pltpu.VMEM((num_i,), jnp.float32),
        ],
    )
    out = f(inp_flat, w_chunks)
    return out.reshape(NW, num_b, L).transpose(1, 0, 2).reshape(num_b, num_o)


# speedup vs baseline: 1.2470x; 1.0073x over previous
import functools

import jax
import jax.numpy as jnp
from jax import lax
from jax.experimental import pallas as pl
from jax.experimental.pallas import tpu as pltpu
from jax.experimental.pallas import tpu_sc as plsc

K = 8
L = 16
NUM_CORES = 2
NUM_SUBCORES = 16
NW = NUM_CORES * NUM_SUBCORES


_SORT8 = ((0, 1), (2, 3), (4, 5), (6, 7),
          (0, 2), (1, 3), (4, 6), (5, 7),
          (1, 2), (5, 6),
          (0, 4), (1, 5), (2, 6), (3, 7),
          (2, 4), (3, 5),
          (1, 2), (3, 4), (5, 6))
_BITONIC8 = ((0, 4), (1, 5), (2, 6), (3, 7),
             (0, 2), (1, 3), (4, 6), (5, 7),
             (0, 1), (2, 3), (4, 5), (6, 7))


def _apply_net(net, v):
    v = list(v)
    for a, b in net:
        lo = jnp.minimum(v[a], v[b])
        hi = jnp.maximum(v[a], v[b])
        v[a], v[b] = lo, hi
    return v


def _merge_bottom8(S, C):
    t = [jnp.minimum(S[i], C[K - 1 - i]) for i in range(K)]
    return _apply_net(_BITONIC8, t)


def _bcast_lane(v, j):
    idx = jnp.full((L,), j, dtype=jnp.int32)
    return v.at[idx].get(mode="promise_in_bounds")


def _sc_kernel(num_b, num_i, inp_hbm, w_hbm, out_hbm,
               inp_v, w_v, out_v, pw_v, mw_v, pi_v, mi_v):
    wid = lax.axis_index("s") * NUM_CORES + lax.axis_index("c")
    chunk_w = num_i * L
    chunk_o = num_b * L
    pltpu.sync_copy(inp_hbm, inp_v)
    pltpu.sync_copy(w_hbm.at[pl.ds(wid * chunk_w, chunk_w)], w_v)

    inf = jnp.full((L,), jnp.inf, dtype=jnp.float32)

    def w_body(i, carry):
        w = w_v[pl.ds(i * L, L)]
        pw = jnp.maximum(w, 0.0)
        pw_v[pl.ds(i * L, L)] = pw
        mw_v[pl.ds(i * L, L)] = pw - w
        return carry

    lax.fori_loop(0, num_i, w_body, 0)

    def b_body(b, carry):
        base = b * num_i

        def x_body(k, carry2):
            xv = inp_v[pl.ds(base + k * L, L)]
            pi_v[pl.ds(k * L, L)] = jnp.maximum(xv + 4.0, 0.0)
            mi_v[pl.ds(k * L, L)] = jnp.maximum(4.0 - xv, 0.0)
            return carry2

        lax.fori_loop(0, num_i // L, x_body, 0)

        def blk_body(k, S):
            piv = pi_v[pl.ds(k * L, L)]
            miv = mi_v[pl.ds(k * L, L)]
            Tp, Tm = [], []
            for q in range(L // 4):
                Cp, Cm = [], []
                for j in range(4 * q, 4 * q + 4):
                    pi = _bcast_lane(piv, j)
                    mi = _bcast_lane(miv, j)
                    pw = pw_v[pl.ds((k * L + j) * L, L)]
                    mw = mw_v[pl.ds((k * L + j) * L, L)]
                    Cp += [pi + pw, mi + mw]
                    Cm += [pi + mw, mi + pw]
                Tp.append(_apply_net(_SORT8, Cp))
                Tm.append(_apply_net(_SORT8, Cm))
            Tp1 = _merge_bottom8(Tp[0], Tp[1])
            Tp2 = _merge_bottom8(Tp[2], Tp[3])
            Tm1 = _merge_bottom8(Tm[0], Tm[1])
            Tm2 = _merge_bottom8(Tm[2], Tm[3])
            Sp = _merge_bottom8(list(S[:K]), _merge_bottom8(Tp1, Tp2))
            Sm = _merge_bottom8(list(S[K:]), _merge_bottom8(Tm1, Tm2))
            return tuple(Sp) + tuple(Sm)

        S = lax.fori_loop(0, num_i // L, blk_body, (inf,) * (2 * K))
        SP, SM = S[:K], S[K:]
        resP = SP[0]
        for j in range(1, K):
            resP = resP + SP[j]
        resM = SM[0]
        for j in range(1, K):
            resM = resM + SM[j]
        out_v[pl.ds(b * L, L)] = (resP - resM) * (1.0 / K)
        return carry

    lax.fori_loop(0, num_b, b_body, 0)
    pltpu.sync_copy(out_v, out_hbm.at[pl.ds(wid * chunk_o, chunk_o)])


def _tc_body(x_ref, w_ref, out_ref, pw_s, mw_s, pib_s, mib_s):
    num_i = x_ref.shape[1]
    oj = pl.program_id(1)

    @pl.when(oj == 0)
    def _():
        x = x_ref[...]
        pi_full = jnp.maximum(x + 4.0, 0.0)
        mi_full = jnp.maximum(4.0 - x, 0.0)
        lane = lax.broadcasted_iota(jnp.int32, (8, num_i), 1)

        def build(i, c):
            bp = jnp.max(jnp.where(lane == i, pi_full, -jnp.inf),
                         axis=1, keepdims=True)
            bm = jnp.max(jnp.where(lane == i, mi_full, -jnp.inf),
                         axis=1, keepdims=True)
            pib_s[i] = jnp.broadcast_to(bp, (8, 128))
            mib_s[i] = jnp.broadcast_to(bm, (8, 128))
            return c

        lax.fori_loop(0, num_i, build, 0)

    w = w_ref[...]
    pw_full = jnp.maximum(w, 0.0)
    pw_s[...] = pw_full
    mw_s[...] = pw_full - w
    inf = jnp.full((8, 128), jnp.inf, dtype=jnp.float32)

    def half(k, q0):
        T = []
        for q in (q0, q0 + 1):
            Cp, Cm = [], []
            for j in range(4 * q, 4 * q + 4):
                i = k * L + j
                pi = pib_s[i]
                mi = mib_s[i]
                pw = jnp.broadcast_to(pw_s[pl.ds(i, 1), :], (8, 128))
                mw = jnp.broadcast_to(mw_s[pl.ds(i, 1), :], (8, 128))
                Cp += [pi + pw, mi + mw]
                Cm += [pi + mw, mi + pw]
            T.append((_apply_net(_SORT8, Cp), _apply_net(_SORT8, Cm)))
        return (_merge_bottom8(T[0][0], T[1][0]),
                _merge_bottom8(T[0][1], T[1][1]))

    def tree16(k):
        Tp1, Tm1 = half(k, 0)
        Tp2, Tm2 = half(k, 2)
        return _merge_bottom8(Tp1, Tp2), _merge_bottom8(Tm1, Tm2)

    def blk_body(k2, S):
        Hp1, Hm1 = tree16(2 * k2)
        Hp2, Hm2 = tree16(2 * k2 + 1)
        Sp = _merge_bottom8(list(S[:K]), _merge_bottom8(Hp1, Hp2))
        Sm = _merge_bottom8(list(S[K:]), _merge_bottom8(Hm1, Hm2))
        return tuple(Sp) + tuple(Sm)

    S = lax.fori_loop(0, num_i // (2 * L), blk_body, (inf,) * (2 * K))
    resP = S[0]
    for j in range(1, K):
        resP = resP + S[j]
    resM = S[K]
    for j in range(1, K):
        resM = resM + S[K + j]
    out_ref[...] = (resP - resM) * (1.0 / K)


def _tc_part(x_tc, weight):
    b_tc, num_i = x_tc.shape
    _, num_o = weight.shape
    return pl.pallas_call(
        _tc_body,
        grid=(b_tc // 8, num_o // 128),
        in_specs=[
            pl.BlockSpec((8, num_i), lambda bi, oj: (bi, 0)),
            pl.BlockSpec((num_i, 128), lambda bi, oj: (0, oj)),
        ],
        out_specs=pl.BlockSpec((8, 128), lambda bi, oj: (bi, oj)),
        out_shape=jax.ShapeDtypeStruct((b_tc, num_o), jnp.float32),
        scratch_shapes=[
            pltpu.VMEM((num_i, 128), jnp.float32),
            pltpu.VMEM((num_i, 128), jnp.float32),
            pltpu.VMEM((num_i, 8, 128), jnp.float32),
            pltpu.VMEM((num_i, 8, 128), jnp.float32),
        ],
    )(x_tc, weight)


B_SC = 72


def kernel(inputp, weight):
    num_b, num_i = inputp.shape
    _, num_o = weight.shape
    b_sc = B_SC if num_b > B_SC else num_b
    if b_sc < num_b:
        out_tc = _tc_part(inputp[b_sc:], weight)
        out_sc = _sc_part(inputp[:b_sc], weight)
        return jnp.concatenate([out_sc, out_tc], axis=0)
    return _sc_part(inputp, weight)


def _sc_part(inputp, weight):
    num_b, num_i = inputp.shape
    _, num_o = weight.shape
    w_chunks = weight.reshape(num_i, NW, L).transpose(1, 0, 2).reshape(-1)
    inp_flat = inputp.reshape(-1)
    mesh = plsc.VectorSubcoreMesh(
        core_axis_name="c", subcore_axis_name="s",
        num_cores=NUM_CORES, num_subcores=NUM_SUBCORES)
    f = pl.kernel(
        functools.partial(_sc_kernel, num_b, num_i),
        out_type=jax.ShapeDtypeStruct((NW * num_b * L,), jnp.float32),
        mesh=mesh,
        scratch_types=[
            pltpu.VMEM((num_b * num_i,), jnp.float32),
            pltpu.VMEM((num_i * L,), jnp.float32),
            pltpu.VMEM((num_b * L,), jnp.float32),
            pltpu.VMEM((num_i * L,), jnp.float32),
            pltpu.VMEM((num_i * L,), jnp.float32),
            pltpu.VMEM((num_i,), jnp.float32),
            pltpu.VMEM((num_i,), jnp.float32),
        ],
    )
    out = f(inp_flat, w_chunks)
    return out.reshape(NW, num_b, L).transpose(1, 0, 2).reshape(num_b, num_o)
